# baseline (speedup 1.0000x reference)
import functools

import jax
import jax.numpy as jnp
from jax import lax
from jax.experimental import pallas as pl
from jax.experimental.pallas import tpu as pltpu
from jax.experimental.pallas import tpu_sc as plsc

_D = 64
_NW = 32
_CHUNK = 128
_EPS = 1e-5


def _rsqrt(x):
    i = plsc.bitcast(x, jnp.int32)
    i = jnp.int32(0x5F3759DF) - lax.shift_right_arithmetic(i, 1)
    y = plsc.bitcast(i, jnp.float32)
    h = x * 0.5
    for _ in range(3):
        y = y * (1.5 - h * y * y)
    return y


def _make_kernel(n_chunks):
    mesh = plsc.VectorSubcoreMesh(core_axis_name="c", subcore_axis_name="s")

    @functools.partial(
        pl.kernel,
        out_type=jax.ShapeDtypeStruct((_NW, n_chunks, _CHUNK, _D), jnp.float32),
        mesh=mesh,
        compiler_params=pltpu.CompilerParams(
            needs_layout_passes=False, use_tc_tiling_on_sc=False
        ),
        scratch_types=[
            pltpu.VMEM((n_chunks, _CHUNK), jnp.int32),
            pltpu.VMEM((_CHUNK, _D), jnp.float32),
            pltpu.VMEM((_D,), jnp.float32),
            pltpu.VMEM((_D,), jnp.float32),
            pltpu.SemaphoreType.DMA,
        ],
    )
    def kern(idx_hbm, table_hbm, gamma_hbm, beta_hbm, out_hbm,
             idx_v, rows_v, gamma_v, beta_v, gsem):
        wid = lax.axis_index("s") * 2 + lax.axis_index("c")
        pltpu.sync_copy(gamma_hbm, gamma_v)
        pltpu.sync_copy(beta_hbm, beta_v)
        pltpu.sync_copy(idx_hbm.at[wid], idx_v)
        lanes = lax.iota(jnp.int32, 16)

        def chunk_body(c, carry):
            pltpu.async_copy(table_hbm.at[idx_v.at[c]], rows_v, gsem).wait()

            def group_body(g, carry2):
                row_ids = lanes + g * 16
                sum_v = jnp.zeros((16,), jnp.float32)
                sq_v = jnp.zeros((16,), jnp.float32)
                for j in range(_D):
                    cj = jnp.full((16,), j, jnp.int32)
                    col = plsc.load_gather(rows_v, [row_ids, cj])
                    sum_v = sum_v + col
                    sq_v = sq_v + col * col
                mean = sum_v * (1.0 / _D)
                var = sq_v * (1.0 / _D) - mean * mean
                rstd = _rsqrt(var + _EPS)
                mrs = mean * rstd
                for j in range(_D):
                    cj = jnp.full((16,), j, jnp.int32)
                    col = plsc.load_gather(rows_v, [row_ids, cj])
                    gj = plsc.load_gather(gamma_v, [cj])
                    bj = plsc.load_gather(beta_v, [cj])
                    o = (col * rstd - mrs) * gj + bj
                    plsc.store_scatter(rows_v, [row_ids, cj], o)
                return carry2

            lax.fori_loop(0, _CHUNK // 16, group_body, 0)
            pltpu.sync_copy(rows_v, out_hbm.at[wid, c])
            return carry

        lax.fori_loop(0, n_chunks, chunk_body, 0)

    return kern


@jax.jit
def kernel(place_ids, table, gamma, beta):
    batch, hist = place_ids.shape
    total = batch * hist
    n_chunks = total // (_NW * _CHUNK)
    idx = place_ids.astype(jnp.int32).reshape(_NW, n_chunks, _CHUNK)
    out = _make_kernel(n_chunks)(idx, table, gamma, beta)
    return out.reshape(batch, hist, _D)


# speedup vs baseline: 2.6175x; 2.6175x over previous
import functools

import jax
import jax.numpy as jnp
from jax import lax
from jax.experimental import pallas as pl
from jax.experimental.pallas import tpu as pltpu
from jax.experimental.pallas import tpu_sc as plsc

_D = 64
_NW = 32
_CHUNK = 128
_EPS = 1e-5


def _rsqrt(x):
    i = plsc.bitcast(x, jnp.int32)
    i = jnp.int32(0x5F3759DF) - lax.shift_right_arithmetic(i, 1)
    y = plsc.bitcast(i, jnp.float32)
    h = x * 0.5
    for _ in range(3):
        y = y * (1.5 - h * y * y)
    return y


def _make_kernel(n_chunks):
    mesh = plsc.VectorSubcoreMesh(core_axis_name="c", subcore_axis_name="s")

    @functools.partial(
        pl.kernel,
        out_type=jax.ShapeDtypeStruct((_NW, n_chunks, _CHUNK, _D), jnp.float32),
        mesh=mesh,
        compiler_params=pltpu.CompilerParams(
            needs_layout_passes=False, use_tc_tiling_on_sc=False
        ),
        scratch_types=[
            pltpu.VMEM((n_chunks, _CHUNK), jnp.int32),
            pltpu.VMEM((_CHUNK, _D), jnp.float32),
            pltpu.VMEM((_D,), jnp.float32),
            pltpu.VMEM((_D,), jnp.float32),
            pltpu.SemaphoreType.DMA,
        ],
    )
    def kern(idx_hbm, table_hbm, gamma_hbm, beta_hbm, out_hbm,
             idx_v, rows_v, gamma_v, beta_v, gsem):
        wid = lax.axis_index("s") * 2 + lax.axis_index("c")
        pltpu.sync_copy(gamma_hbm, gamma_v)
        pltpu.sync_copy(beta_hbm, beta_v)
        pltpu.sync_copy(idx_hbm.at[wid], idx_v)
        gam = [gamma_v[pl.ds(16 * i, 16)] for i in range(_D // 16)]
        bet = [beta_v[pl.ds(16 * i, 16)] for i in range(_D // 16)]

        def chunk_body(c, carry):
            pltpu.async_copy(table_hbm.at[idx_v.at[c]], rows_v, gsem).wait()

            @plsc.parallel_loop(0, _CHUNK, unroll=4)
            def row_body(r):
                vs = [rows_v[r, pl.ds(16 * i, 16)] for i in range(_D // 16)]
                s = jnp.sum(vs[0] + vs[1] + vs[2] + vs[3])
                q = jnp.sum(
                    vs[0] * vs[0] + vs[1] * vs[1] + vs[2] * vs[2] + vs[3] * vs[3]
                )
                mean = s * (1.0 / _D)
                var = q * (1.0 / _D) - mean * mean
                var_v = jnp.full((16,), var, jnp.float32) + _EPS
                rstd = _rsqrt(var_v)
                mrs = jnp.full((16,), mean, jnp.float32) * rstd
                for i in range(_D // 16):
                    o = (vs[i] * rstd - mrs) * gam[i] + bet[i]
                    rows_v[r, pl.ds(16 * i, 16)] = o

            pltpu.sync_copy(rows_v, out_hbm.at[wid, c])
            return carry

        lax.fori_loop(0, n_chunks, chunk_body, 0)

    return kern


@jax.jit
def kernel(place_ids, table, gamma, beta):
    batch, hist = place_ids.shape
    total = batch * hist
    n_chunks = total // (_NW * _CHUNK)
    idx = place_ids.astype(jnp.int32).reshape(_NW, n_chunks, _CHUNK)
    out = _make_kernel(n_chunks)(idx, table, gamma, beta)
    return out.reshape(batch, hist, _D)


# speedup vs baseline: 3.1580x; 1.2065x over previous
import functools

import jax
import jax.numpy as jnp
from jax import lax
from jax.experimental import pallas as pl
from jax.experimental.pallas import tpu as pltpu
from jax.experimental.pallas import tpu_sc as plsc

_D = 64
_NW = 32
_CHUNK = 128
_NBUF = 4
_LOOK = 2
_EPS = 1e-5


def _rsqrt(x):
    i = plsc.bitcast(x, jnp.int32)
    i = jnp.int32(0x5F3759DF) - lax.shift_right_arithmetic(i, 1)
    y = plsc.bitcast(i, jnp.float32)
    h = x * 0.5
    for _ in range(3):
        y = y * (1.5 - h * y * y)
    return y


def _make_kernel(n_chunks):
    mesh = plsc.VectorSubcoreMesh(core_axis_name="c", subcore_axis_name="s")

    @functools.partial(
        pl.kernel,
        out_type=jax.ShapeDtypeStruct((_NW, n_chunks, _CHUNK, _D), jnp.float32),
        mesh=mesh,
        compiler_params=pltpu.CompilerParams(
            needs_layout_passes=False, use_tc_tiling_on_sc=False
        ),
        scratch_types=[
            pltpu.VMEM((n_chunks, _CHUNK), jnp.int32),
            [pltpu.VMEM((_CHUNK, _D), jnp.float32) for _ in range(_NBUF)],
            pltpu.VMEM((_D,), jnp.float32),
            pltpu.VMEM((_D,), jnp.float32),
            [pltpu.SemaphoreType.DMA for _ in range(_NBUF)],
            [pltpu.SemaphoreType.DMA for _ in range(_NBUF)],
        ],
    )
    def kern(idx_hbm, table_hbm, gamma_hbm, beta_hbm, out_hbm,
             idx_v, rows, gamma_v, beta_v, gsem, ssem):
        wid = lax.axis_index("s") * 2 + lax.axis_index("c")
        pltpu.sync_copy(gamma_hbm, gamma_v)
        pltpu.sync_copy(beta_hbm, beta_v)
        pltpu.sync_copy(idx_hbm.at[wid], idx_v)
        gam = [gamma_v[pl.ds(16 * i, 16)] for i in range(_D // 16)]
        bet = [beta_v[pl.ds(16 * i, 16)] for i in range(_D // 16)]

        def gather(c, b):
            return pltpu.make_async_copy(
                table_hbm.at[idx_v.at[c]], rows[b], gsem[b]
            )

        def scatter(c, b):
            return pltpu.make_async_copy(rows[b], out_hbm.at[wid, c], ssem[b])

        def compute(b):
            @plsc.parallel_loop(0, _CHUNK, unroll=4)
            def row_body(r):
                vs = [rows[b][r, pl.ds(16 * i, 16)] for i in range(_D // 16)]
                s = jnp.sum(vs[0] + vs[1] + vs[2] + vs[3])
                q = jnp.sum(
                    vs[0] * vs[0] + vs[1] * vs[1] + vs[2] * vs[2] + vs[3] * vs[3]
                )
                mean = s * (1.0 / _D)
                var = q * (1.0 / _D) - mean * mean
                var_v = jnp.full((16,), var, jnp.float32) + _EPS
                rstd = _rsqrt(var_v)
                mrs = jnp.full((16,), mean, jnp.float32) * rstd
                for i in range(_D // 16):
                    o = (vs[i] * rstd - mrs) * gam[i] + bet[i]
                    rows[b][r, pl.ds(16 * i, 16)] = o

        gather(0, 0).start()
        gather(1, 1).start()

        def ring_body(cc, carry):
            for b in range(_NBUF):
                c = cc * _NBUF + b
                nb = (b + _LOOK) % _NBUF

                @pl.when(c + _LOOK < n_chunks)
                def _():
                    @pl.when(c >= _NBUF - _LOOK)
                    def _():
                        scatter(c - (_NBUF - _LOOK), nb).wait()

                    gather(c + _LOOK, nb).start()

                gather(c, b).wait()
                compute(b)
                scatter(c, b).start()
            return carry

        lax.fori_loop(0, n_chunks // _NBUF, ring_body, 0)
        for b in range(_NBUF):
            scatter(n_chunks - _NBUF + b, b).wait()

    return kern


@jax.jit
def kernel(place_ids, table, gamma, beta):
    batch, hist = place_ids.shape
    total = batch * hist
    n_chunks = total // (_NW * _CHUNK)
    idx = place_ids.astype(jnp.int32).reshape(_NW, n_chunks, _CHUNK)
    out = _make_kernel(n_chunks)(idx, table, gamma, beta)
    return out.reshape(batch, hist, _D)
